# baseline (speedup 1.0000x reference)
import functools

import jax
import jax.numpy as jnp
from jax import lax
from jax.experimental import pallas as pl
from jax.experimental.pallas import tpu as pltpu
from jax.experimental.pallas import tpu_sc as plsc

N_EMB = 32
SEQ = 50
NROWS = 16384
B_TOTAL = NROWS * SEQ
NBT = B_TOTAL // 128

_info = plsc.get_sparse_core_info()
NUM_CORES = _info.num_cores
NUM_SUBCORES = _info.num_subcores
NW = NUM_CORES * NUM_SUBCORES
B_PER_W = B_TOTAL // NW
CHUNK = 512
BLKS = CHUNK // 128
NCHUNK = B_PER_W // CHUNK
OUTER = NCHUNK // 2


@functools.partial(
    pl.kernel,
    mesh=plsc.VectorSubcoreMesh(core_axis_name="c", subcore_axis_name="s"),
    out_type=jax.ShapeDtypeStruct((4, NBT, 8, 128), jnp.float32),
    scratch_types=[
        pltpu.VMEM((B_PER_W,), jnp.int32),
        pltpu.VMEM((CHUNK, N_EMB), jnp.float32),
        pltpu.VMEM((CHUNK, N_EMB), jnp.float32),
        pltpu.VMEM((4, BLKS, 8, 128), jnp.float32),
        pltpu.VMEM((4, BLKS, 8, 128), jnp.float32),
        pltpu.SemaphoreType.DMA,
        pltpu.SemaphoreType.DMA,
        pltpu.SemaphoreType.DMA,
        pltpu.SemaphoreType.DMA,
    ],
    compiler_params=pltpu.CompilerParams(use_tc_tiling_on_sc=False,
                                         needs_layout_passes=False),
)
def _gather_all(words_hbm, table_hbm, out_hbm, idx_v, grow0, grow1,
                tiles0, tiles1, g0, g1, w0, w1):
    wid = lax.axis_index("s") * NUM_CORES + lax.axis_index("c")
    base = wid * B_PER_W
    bt_base = wid * (B_PER_W // 128)
    grow = (grow0, grow1)
    tiles = (tiles0, tiles1)
    gsem = (g0, g1)
    wsem = (w0, w1)

    pltpu.sync_copy(words_hbm.at[pl.ds(base, B_PER_W)], idx_v)

    lane = jnp.arange(16, dtype=jnp.int32)
    dt_lo = lane >> 3
    di_lo = lane & 7
    dt_hi = (lane + 16) >> 3
    di_hi = (lane + 16) & 7

    def gather_start(c, b):
        pltpu.async_copy(table_hbm.at[idx_v.at[pl.ds(c * CHUNK, CHUNK)]],
                         grow[b], gsem[b])

    def gather_wait(c, b):
        pltpu.make_async_copy(table_hbm.at[idx_v.at[pl.ds(c * CHUNK, CHUNK)]],
                              grow[b], gsem[b]).wait()

    def transpose_chunk(b):
        gr = grow[b]
        tl = tiles[b]

        def tbody(k, carry):
            blk = k >> 7
            bi = k & 127
            i_blk = jnp.full((16,), blk, dtype=jnp.int32)
            i_bi = jnp.full((16,), bi, dtype=jnp.int32)
            x_lo = gr[k, pl.ds(0, 16)]
            x_hi = gr[k, pl.ds(16, 16)]
            plsc.store_scatter(tl, [dt_lo, i_blk, di_lo, i_bi], x_lo)
            plsc.store_scatter(tl, [dt_hi, i_blk, di_hi, i_bi], x_hi)
            return carry

        lax.fori_loop(0, CHUNK, tbody, 0)

    def wb_start(c, b):
        for dt in range(4):
            pltpu.async_copy(tiles[b].at[dt],
                             out_hbm.at[dt, pl.ds(bt_base + c * BLKS, BLKS)],
                             wsem[b])

    def wb_wait(c, b):
        for dt in range(4):
            pltpu.make_async_copy(
                tiles[b].at[dt],
                out_hbm.at[dt, pl.ds(bt_base + c * BLKS, BLKS)],
                wsem[b]).wait()

    gather_start(0, 0)

    def body(t, carry):
        c0 = t * 2
        gather_wait(c0, 0)
        gather_start(c0 + 1, 1)
        transpose_chunk(0)
        wb_start(c0, 0)
        wb_wait(c0, 0)

        @pl.when(t + 1 < OUTER)
        def _():
            gather_start(c0 + 2, 0)

        gather_wait(c0 + 1, 1)
        transpose_chunk(1)
        wb_start(c0 + 1, 1)
        wb_wait(c0 + 1, 1)
        return carry

    lax.fori_loop(0, OUTER, body, 0)


def kernel(words, table):
    flat = words.reshape(B_TOTAL).astype(jnp.int32)
    out_t4 = _gather_all(flat, table)
    o = out_t4.transpose(0, 2, 1, 3).reshape(N_EMB, B_TOTAL).T
    return o.reshape(NROWS, SEQ, N_EMB)
